# Initial kernel scaffold; baseline (speedup 1.0000x reference)
import jax
import jax.numpy as jnp
from jax.experimental import pallas as pl


def kernel(x_src, x_dst, edge_src, edge_dst):
    raise NotImplementedError("write your pallas kernel here")


# speedup vs baseline: 9.5793x; 9.5793x over previous
import functools

import jax
import jax.numpy as jnp
from jax import lax
from jax.experimental import pallas as pl
from jax.experimental.pallas import tpu as pltpu
from jax.experimental.pallas import tpu_sc as plsc

NC = 2
NS = 16
NW = NC * NS
LANES = 16

ROWS = 8
C = ROWS * 128
GROUPS = C // LANES

F32 = jnp.float32
I32 = jnp.int32

S3 = 3.0 ** 0.5
S5 = 5.0 ** 0.5
S15 = 15.0 ** 0.5
PI = 3.14159265358979
LOGEPS = -27.631021
LN2 = 0.6931471805599453
SQRT2 = 1.4142135


def _edge_kernel_body(nchunk, base_cnt, extra_w,
                      xs_hbm, xd_hbm, es_hbm, ed_hbm,
                      sh_hbm, len_hbm, cut_hbm, log_hbm,
                      idx_s, idx_d, rows_s, rows_d,
                      shb, lb, cb, gb, sem_s, sem_d):
    wid = lax.axis_index("s") * NC + lax.axis_index("c")
    my_cnt = base_cnt + jnp.where(wid < extra_w, 1, 0).astype(I32)

    iota = lax.iota(I32, 16)
    ones = jnp.full((16,), 1.0, F32)
    cols = [jnp.full((16,), c, I32) for c in range(9)]

    def chunk_body(i, carry):
        c = wid + i * NW
        rbase = pl.multiple_of(c * ROWS, 8)
        ebase = pl.multiple_of(c * C, 128)
        pltpu.sync_copy(es_hbm.at[pl.ds(rbase, ROWS)], idx_s)
        pltpu.sync_copy(ed_hbm.at[pl.ds(rbase, ROWS)], idx_d)
        handles = []
        for j in range(ROWS):
            handles.append(pltpu.async_copy(
                xs_hbm.at[idx_s.at[j]], rows_s.at[pl.ds(j * 128, 128)], sem_s))
            handles.append(pltpu.async_copy(
                xd_hbm.at[idx_d.at[j]], rows_d.at[pl.ds(j * 128, 128)], sem_d))
        for h in handles:
            h.wait()

        def group(g, gcarry):
            row = iota + g * LANES
            sx = plsc.load_gather(rows_s, [row, cols[0]])
            sy = plsc.load_gather(rows_s, [row, cols[1]])
            sz = plsc.load_gather(rows_s, [row, cols[2]])
            dx = plsc.load_gather(rows_d, [row, cols[0]])
            dy = plsc.load_gather(rows_d, [row, cols[1]])
            dz = plsc.load_gather(rows_d, [row, cols[2]])
            vx = sx - dx
            vy = sy - dy
            vz = sz - dz
            r2 = vx * vx + vy * vy + vz * vz
            bi = lax.bitcast_convert_type(r2, I32)
            y = lax.bitcast_convert_type(
                jnp.int32(0x5F3759DF) - lax.shift_right_logical(bi, 1), F32)
            y = y * (1.5 - 0.5 * r2 * y * y)
            y = y * (1.5 - 0.5 * r2 * y * y)
            y = y * (1.5 - 0.5 * r2 * y * y)
            L = r2 * y
            ux = vx * y
            uy = vy * y
            uz = vz * y
            xx = ux * ux
            yy = uy * uy
            zz = uz * uz
            zarg = (0.5 * PI) * (5.0 - L)
            z2 = zarg * zarg
            p = -1.0 / 39916800.0
            p = p * z2 + 1.0 / 362880.0
            p = p * z2 - 1.0 / 5040.0
            p = p * z2 + 1.0 / 120.0
            p = p * z2 - 1.0 / 6.0
            p = p * z2 + 1.0
            h = zarg * p
            decay = h * h
            lt4 = L < 4.0
            gt5 = L > 5.0
            cut = jnp.where(lt4, 1.0, jnp.where(gt5, 0.0, decay))
            cut = jnp.maximum(cut, 1e-12)
            ib = lax.bitcast_convert_type(cut, I32)
            e = lax.shift_right_logical(ib, 23) - 127
            m = lax.bitcast_convert_type(
                (ib & jnp.int32(0x007FFFFF)) | jnp.int32(0x3F800000), F32)
            big = m > SQRT2
            m = jnp.where(big, 0.5 * m, m)
            ef = (e + jnp.where(big, 1, 0)).astype(F32)
            w = (m - 1.0) / (m + 1.0)
            w2 = w * w
            q = 2.0 / 9.0
            q = q * w2 + 2.0 / 7.0
            q = q * w2 + 2.0 / 5.0
            q = q * w2 + 2.0 / 3.0
            q = q * w2 + 2.0
            lg = ef * LN2 + w * q
            lg = jnp.where(lt4, 0.0, jnp.where(gt5, LOGEPS, lg))
            plsc.store_scatter(shb, [row, cols[0]], ones)
            plsc.store_scatter(shb, [row, cols[1]], S3 * ux)
            plsc.store_scatter(shb, [row, cols[2]], S3 * uy)
            plsc.store_scatter(shb, [row, cols[3]], S3 * uz)
            plsc.store_scatter(shb, [row, cols[4]], S15 * ux * uz)
            plsc.store_scatter(shb, [row, cols[5]], S15 * ux * uy)
            plsc.store_scatter(shb, [row, cols[6]], S5 * (yy - 0.5 * (xx + zz)))
            plsc.store_scatter(shb, [row, cols[7]], S15 * uy * uz)
            plsc.store_scatter(shb, [row, cols[8]], (0.5 * S15) * (zz - xx))
            lb[pl.ds(g * LANES, LANES)] = L
            cb[pl.ds(g * LANES, LANES)] = cut
            gb[pl.ds(g * LANES, LANES)] = lg
            return gcarry

        lax.fori_loop(0, GROUPS, group, 0)
        pltpu.sync_copy(shb, sh_hbm.at[pl.ds(ebase, C)])
        pltpu.sync_copy(lb, len_hbm.at[pl.ds(ebase, C)])
        pltpu.sync_copy(cb, cut_hbm.at[pl.ds(ebase, C)])
        pltpu.sync_copy(gb, log_hbm.at[pl.ds(ebase, C)])
        return carry

    lax.fori_loop(0, my_cnt, chunk_body, 0)


def kernel(x_src, x_dst, edge_src, edge_dst):
    n = x_src.shape[0]
    e = edge_src.shape[0]
    assert e % 128 == 0
    nrows = e // 128
    nchunk = nrows // ROWS
    assert nchunk * ROWS == nrows
    base_cnt = nchunk // NW
    extra_w = nchunk - base_cnt * NW

    xs4 = jnp.concatenate([x_src, jnp.zeros((n, 5), F32)], axis=1)
    xd4 = jnp.concatenate([x_dst, jnp.zeros((n, 5), F32)], axis=1)
    es2 = edge_src.reshape(nrows, 128)
    ed2 = edge_dst.reshape(nrows, 128)

    mesh = plsc.VectorSubcoreMesh(core_axis_name="c", subcore_axis_name="s")
    run = pl.kernel(
        functools.partial(_edge_kernel_body, nchunk, base_cnt, extra_w),
        out_type=(
            jax.ShapeDtypeStruct((e, 9), F32),
            jax.ShapeDtypeStruct((e,), F32),
            jax.ShapeDtypeStruct((e,), F32),
            jax.ShapeDtypeStruct((e,), F32),
        ),
        mesh=mesh,
        compiler_params=pltpu.CompilerParams(
            needs_layout_passes=False, use_tc_tiling_on_sc=False),
        scratch_types=[
            pltpu.VMEM((ROWS, 128), I32),
            pltpu.VMEM((ROWS, 128), I32),
            pltpu.VMEM((C, 8), F32),
            pltpu.VMEM((C, 8), F32),
            pltpu.VMEM((C, 9), F32),
            pltpu.VMEM((C,), F32),
            pltpu.VMEM((C,), F32),
            pltpu.VMEM((C,), F32),
            pltpu.SemaphoreType.DMA,
            pltpu.SemaphoreType.DMA,
        ],
    )
    sh, length, cut, lg = run(xs4, xd4, es2, ed2)
    return (edge_src, edge_dst, sh, length, cut, lg)


# speedup vs baseline: 9.7304x; 1.0158x over previous
import functools

import jax
import jax.numpy as jnp
from jax import lax
from jax.experimental import pallas as pl
from jax.experimental.pallas import tpu as pltpu
from jax.experimental.pallas import tpu_sc as plsc

NC = 2
NS = 16
NW = NC * NS
LANES = 16

ROWS = 8
C = ROWS * 128
GROUPS = C // LANES

F32 = jnp.float32
I32 = jnp.int32

S3 = 3.0 ** 0.5
S5 = 5.0 ** 0.5
S15 = 15.0 ** 0.5
PI = 3.14159265358979
LOGEPS = -27.631021
LN2 = 0.6931471805599453
SQRT2 = 1.4142135


def _edge_kernel_body(nchunk, base_cnt, extra_w,
                      tbl_hbm, es_hbm, ed_hbm,
                      sh_hbm, len_hbm, cut_hbm, log_hbm,
                      idx_s, idx_d, rows_s, rows_d,
                      shb, lb, cb, gb, sem_s, sem_d):
    wid = lax.axis_index("s") * NC + lax.axis_index("c")
    my_cnt = base_cnt + jnp.where(wid < extra_w, 1, 0).astype(I32)

    iota = lax.iota(I32, 16)
    ones = jnp.full((16,), 1.0, F32)
    cols = [jnp.full((16,), c, I32) for c in range(9)]

    def chunk_body(i, carry):
        c = wid + i * NW
        rbase = pl.multiple_of(c * ROWS, 8)
        ebase = pl.multiple_of(c * C, 128)
        pltpu.sync_copy(es_hbm.at[pl.ds(rbase, ROWS)], idx_s)
        pltpu.sync_copy(ed_hbm.at[pl.ds(rbase, ROWS)], idx_d)
        handles = []
        for j in range(ROWS):
            handles.append(pltpu.async_copy(
                tbl_hbm.at[idx_s.at[j]], rows_s.at[pl.ds(j * 128, 128)], sem_s))
            handles.append(pltpu.async_copy(
                tbl_hbm.at[idx_d.at[j]], rows_d.at[pl.ds(j * 128, 128)], sem_d))
        for h in handles:
            h.wait()

        def group(g, gcarry):
            row = iota + g * LANES
            sx = plsc.load_gather(rows_s, [row, cols[0]])
            sy = plsc.load_gather(rows_s, [row, cols[1]])
            sz = plsc.load_gather(rows_s, [row, cols[2]])
            dx = plsc.load_gather(rows_d, [row, cols[3]])
            dy = plsc.load_gather(rows_d, [row, cols[4]])
            dz = plsc.load_gather(rows_d, [row, cols[5]])
            vx = sx - dx
            vy = sy - dy
            vz = sz - dz
            r2 = vx * vx + vy * vy + vz * vz
            bi = lax.bitcast_convert_type(r2, I32)
            y = lax.bitcast_convert_type(
                jnp.int32(0x5F3759DF) - lax.shift_right_logical(bi, 1), F32)
            y = y * (1.5 - 0.5 * r2 * y * y)
            y = y * (1.5 - 0.5 * r2 * y * y)
            y = y * (1.5 - 0.5 * r2 * y * y)
            L = r2 * y
            ux = vx * y
            uy = vy * y
            uz = vz * y
            xx = ux * ux
            yy = uy * uy
            zz = uz * uz
            zarg = (0.5 * PI) * (5.0 - L)
            z2 = zarg * zarg
            p = -1.0 / 39916800.0
            p = p * z2 + 1.0 / 362880.0
            p = p * z2 - 1.0 / 5040.0
            p = p * z2 + 1.0 / 120.0
            p = p * z2 - 1.0 / 6.0
            p = p * z2 + 1.0
            h = zarg * p
            decay = h * h
            lt4 = L < 4.0
            gt5 = L > 5.0
            cut = jnp.where(lt4, 1.0, jnp.where(gt5, 0.0, decay))
            cut = jnp.maximum(cut, 1e-12)
            ib = lax.bitcast_convert_type(cut, I32)
            e = lax.shift_right_logical(ib, 23) - 127
            m = lax.bitcast_convert_type(
                (ib & jnp.int32(0x007FFFFF)) | jnp.int32(0x3F800000), F32)
            big = m > SQRT2
            m = jnp.where(big, 0.5 * m, m)
            ef = (e + jnp.where(big, 1, 0)).astype(F32)
            w = (m - 1.0) / (m + 1.0)
            w2 = w * w
            q = 2.0 / 9.0
            q = q * w2 + 2.0 / 7.0
            q = q * w2 + 2.0 / 5.0
            q = q * w2 + 2.0 / 3.0
            q = q * w2 + 2.0
            lg = ef * LN2 + w * q
            lg = jnp.where(lt4, 0.0, jnp.where(gt5, LOGEPS, lg))
            plsc.store_scatter(shb, [row, cols[0]], ones)
            plsc.store_scatter(shb, [row, cols[1]], S3 * ux)
            plsc.store_scatter(shb, [row, cols[2]], S3 * uy)
            plsc.store_scatter(shb, [row, cols[3]], S3 * uz)
            plsc.store_scatter(shb, [row, cols[4]], S15 * ux * uz)
            plsc.store_scatter(shb, [row, cols[5]], S15 * ux * uy)
            plsc.store_scatter(shb, [row, cols[6]], S5 * (yy - 0.5 * (xx + zz)))
            plsc.store_scatter(shb, [row, cols[7]], S15 * uy * uz)
            plsc.store_scatter(shb, [row, cols[8]], (0.5 * S15) * (zz - xx))
            lb[pl.ds(g * LANES, LANES)] = L
            cb[pl.ds(g * LANES, LANES)] = cut
            gb[pl.ds(g * LANES, LANES)] = lg
            return gcarry

        lax.fori_loop(0, GROUPS, group, 0)
        pltpu.sync_copy(shb, sh_hbm.at[pl.ds(ebase, C)])
        pltpu.sync_copy(lb, len_hbm.at[pl.ds(ebase, C)])
        pltpu.sync_copy(cb, cut_hbm.at[pl.ds(ebase, C)])
        pltpu.sync_copy(gb, log_hbm.at[pl.ds(ebase, C)])
        return carry

    lax.fori_loop(0, my_cnt, chunk_body, 0)


def kernel(x_src, x_dst, edge_src, edge_dst):
    n = x_src.shape[0]
    e = edge_src.shape[0]
    assert e % 128 == 0
    nrows = e // 128
    nchunk = nrows // ROWS
    assert nchunk * ROWS == nrows
    base_cnt = nchunk // NW
    extra_w = nchunk - base_cnt * NW

    pblk = 1000
    assert n % pblk == 0

    def _pad_body(xs_ref, xd_ref, out_ref):
        out_ref[...] = jnp.concatenate(
            [xs_ref[...], xd_ref[...], jnp.zeros((pblk, 2), F32)], axis=1)

    tbl = pl.pallas_call(
        _pad_body,
        grid=(n // pblk,),
        in_specs=[
            pl.BlockSpec((pblk, 3), lambda i: (i, 0)),
            pl.BlockSpec((pblk, 3), lambda i: (i, 0)),
        ],
        out_specs=pl.BlockSpec((pblk, 8), lambda i: (i, 0)),
        out_shape=jax.ShapeDtypeStruct((n, 8), F32),
    )(x_src, x_dst)

    es2 = edge_src.reshape(nrows, 128)
    ed2 = edge_dst.reshape(nrows, 128)

    mesh = plsc.VectorSubcoreMesh(core_axis_name="c", subcore_axis_name="s")
    run = pl.kernel(
        functools.partial(_edge_kernel_body, nchunk, base_cnt, extra_w),
        out_type=(
            jax.ShapeDtypeStruct((e, 9), F32),
            jax.ShapeDtypeStruct((e,), F32),
            jax.ShapeDtypeStruct((e,), F32),
            jax.ShapeDtypeStruct((e,), F32),
        ),
        mesh=mesh,
        compiler_params=pltpu.CompilerParams(
            needs_layout_passes=False, use_tc_tiling_on_sc=False),
        scratch_types=[
            pltpu.VMEM((ROWS, 128), I32),
            pltpu.VMEM((ROWS, 128), I32),
            pltpu.VMEM((C, 8), F32),
            pltpu.VMEM((C, 8), F32),
            pltpu.VMEM((C, 9), F32),
            pltpu.VMEM((C,), F32),
            pltpu.VMEM((C,), F32),
            pltpu.VMEM((C,), F32),
            pltpu.SemaphoreType.DMA,
            pltpu.SemaphoreType.DMA,
        ],
    )
    sh, length, cut, lg = run(tbl, es2, ed2)
    return (edge_src, edge_dst, sh, length, cut, lg)
